# Initial kernel scaffold; baseline (speedup 1.0000x reference)
import functools

import jax
import jax.numpy as jnp
from jax import lax
from jax.experimental import pallas as pl
from jax.experimental.pallas import tpu as pltpu

KB = 100
CC = 25
TOPK = 10
NEG = -1e9

_INTERPRET = False


def _dense_body(slot_ref, intent_ref, wn_cv_ref, nell_cv_ref, wn_idx_ref,
                nell_idx_ref, km_r_ref, km_c_ref, wk_ref, wc_ref, gamma_ref,
                beta_ref, pe_ref, know_ref, ctx_ref):
    D = slot_ref.shape[-1]
    slot = slot_ref[0]
    intent = intent_ref[0]
    wk = wk_ref[...]
    wc = wc_ref[...]
    q = (jnp.dot(slot, wk[:D], preferred_element_type=jnp.float32)
         + jnp.dot(intent, wk[D:], preferred_element_type=jnp.float32))
    q2 = (jnp.dot(slot, wc[:D], preferred_element_type=jnp.float32)
          + jnp.dot(intent, wc[D:], preferred_element_type=jnp.float32))

    wn_cv = wn_cv_ref[0]
    nell_cv = nell_cv_ref[0]
    s_wn = jnp.sum(q[:, None, :] * wn_cv, axis=-1)
    s_nell = jnp.sum(q[:, None, :] * nell_cv, axis=-1)
    scores = jnp.concatenate([s_wn, s_nell], axis=1)
    idx = jnp.concatenate([wn_idx_ref[0], nell_idx_ref[0]], axis=1)
    scores = jnp.where(idx == 0, NEG, scores)

    S = scores.shape[0]
    init = (scores,
            jnp.full((S, 1), NEG, jnp.float32),
            jnp.zeros((S, 1), jnp.int32),
            jnp.zeros((S, 1), jnp.bool_))

    def tk_body(_, carry):
        rem, thresh, taken, done = carry
        m = jnp.max(rem, axis=1, keepdims=True)
        c = jnp.sum((rem == m).astype(jnp.int32), axis=1, keepdims=True)
        new_taken = taken + c
        thresh = jnp.where(done, thresh, m)
        now_done = jnp.logical_or(done, new_taken >= TOPK)
        rem = jnp.where(jnp.logical_and(jnp.logical_not(done), rem == m),
                        -jnp.inf, rem)
        taken = jnp.where(done, taken, new_taken)
        return rem, thresh, taken, now_done

    _, thresh, _, _ = lax.fori_loop(0, TOPK, tk_body, init)

    masked = jnp.where(scores < thresh, NEG, scores)
    mx = jnp.max(masked, axis=1, keepdims=True)
    e = jnp.exp(masked - mx)
    attn = e / jnp.sum(e, axis=1, keepdims=True)

    know = (jnp.sum(attn[:, :CC, None] * wn_cv, axis=1)
            + jnp.sum(attn[:, CC:, None] * nell_cv, axis=1))

    km_c = km_c_ref[0]
    pe = pe_ref[...]
    know = know + jnp.where(km_c == 0, 0.0, pe)

    km_r = km_r_ref[0]
    s2 = lax.dot_general(q2, know, (((1,), (1,)), ((), ())),
                         preferred_element_type=jnp.float32)
    s2 = jnp.where(km_r == 0, NEG, s2)
    mx2 = jnp.max(s2, axis=1, keepdims=True)
    e2 = jnp.exp(s2 - mx2)
    a2 = e2 / jnp.sum(e2, axis=1, keepdims=True)
    ctx = jnp.dot(a2, know, preferred_element_type=jnp.float32)

    mu = jnp.mean(ctx, axis=1, keepdims=True)
    var = jnp.mean((ctx - mu) ** 2, axis=1, keepdims=True)
    ctx = gamma_ref[...] * (ctx - mu) * lax.rsqrt(var + 1e-5) + beta_ref[...]

    know_ref[0] = know
    ctx_ref[0] = ctx


def _dense(slot, intent, wn_cv, nell_cv, wn_idx, nell_idx, km, wk, wc,
           gamma, beta, pe):
    B, S, D = slot.shape
    km_r = km.reshape(B, 1, S)
    km_c = km.reshape(B, S, 1)
    bspec = lambda shp: pl.BlockSpec((1,) + shp, lambda b: (b,) + (0,) * len(shp))
    full = lambda shp: pl.BlockSpec(shp, lambda b: (0,) * len(shp))
    return pl.pallas_call(
        _dense_body,
        grid=(B,),
        in_specs=[
            bspec((S, D)), bspec((S, D)),
            bspec((S, CC, KB)), bspec((S, CC, KB)),
            bspec((S, CC)), bspec((S, CC)),
            bspec((1, S)), bspec((S, 1)),
            full((2 * D, KB)), full((2 * D, KB)),
            full((1, KB)), full((1, KB)), full((S, KB)),
        ],
        out_specs=[bspec((S, KB)), bspec((S, KB))],
        out_shape=[
            jax.ShapeDtypeStruct((B, S, KB), jnp.float32),
            jax.ShapeDtypeStruct((B, S, KB), jnp.float32),
        ],
        interpret=_INTERPRET,
    )(slot, intent, wn_cv, nell_cv, wn_idx, nell_idx, km_r, km_c, wk, wc,
      gamma.reshape(1, KB), beta.reshape(1, KB), pe)


def kernel(intent_features, slot_features, attention_mask, wn_synset_indexes,
           wn_synset_lengths, nell_entity_indexes, nell_entity_lengths,
           wn_table, nell_table, W_k, W_c, gamma, beta, pos_embed):
    B, S, D = slot_features.shape
    wn_cv = jnp.take(wn_table, wn_synset_indexes, axis=0)
    nell_cv = jnp.take(nell_table, nell_entity_indexes, axis=0)
    km = (wn_synset_lengths + nell_entity_lengths).astype(jnp.int32)
    know, ctx = _dense(slot_features, intent_features, wn_cv, nell_cv,
                       wn_synset_indexes.astype(jnp.int32),
                       nell_entity_indexes.astype(jnp.int32),
                       km, W_k, W_c, gamma, beta, pos_embed)
    return (know, ctx)


# speedup vs baseline: 1.9600x; 1.9600x over previous
import functools

import jax
import jax.numpy as jnp
from jax import lax
from jax.experimental import pallas as pl
from jax.experimental.pallas import tpu as pltpu

KB = 100
CC = 25
TOPK = 10
NEG = -1e9

_INTERPRET = False


def _dense_body(slot_ref, intent_ref, wn_cv_ref, nell_cv_ref, wn_idx_ref,
                nell_idx_ref, km_r_ref, km_c_ref, wk_ref, wc_ref, gamma_ref,
                beta_ref, pe_ref, know_ref, ctx_ref):
    D = slot_ref.shape[-1]
    slot = slot_ref[0]
    intent = intent_ref[0]
    wk = wk_ref[...]
    wc = wc_ref[...]
    q = (jnp.dot(slot, wk[:D], preferred_element_type=jnp.float32)
         + jnp.dot(intent, wk[D:], preferred_element_type=jnp.float32))
    q2 = (jnp.dot(slot, wc[:D], preferred_element_type=jnp.float32)
          + jnp.dot(intent, wc[D:], preferred_element_type=jnp.float32))

    wn_cv = wn_cv_ref[0]
    nell_cv = nell_cv_ref[0]
    s_wn = jnp.sum(q[:, None, :] * wn_cv, axis=-1)
    s_nell = jnp.sum(q[:, None, :] * nell_cv, axis=-1)
    scores = jnp.concatenate([s_wn, s_nell], axis=1)
    idx = jnp.concatenate([wn_idx_ref[0], nell_idx_ref[0]], axis=1)
    scores = jnp.where(idx == 0, NEG, scores)

    S = scores.shape[0]
    rem = scores
    thresh = jnp.full((S, 1), NEG, jnp.float32)
    taken = jnp.zeros((S, 1), jnp.int32)
    done = jnp.zeros((S, 1), jnp.bool_)
    for _ in range(TOPK):
        m = jnp.max(rem, axis=1, keepdims=True)
        c = jnp.sum((rem == m).astype(jnp.int32), axis=1, keepdims=True)
        new_taken = taken + c
        thresh = jnp.where(done, thresh, m)
        rem = jnp.where(jnp.logical_and(jnp.logical_not(done), rem == m),
                        -jnp.inf, rem)
        taken = jnp.where(done, taken, new_taken)
        done = jnp.logical_or(done, new_taken >= TOPK)

    masked = jnp.where(scores < thresh, NEG, scores)
    mx = jnp.max(masked, axis=1, keepdims=True)
    e = jnp.exp(masked - mx)
    attn = e / jnp.sum(e, axis=1, keepdims=True)

    know = (jnp.sum(attn[:, :CC, None] * wn_cv, axis=1)
            + jnp.sum(attn[:, CC:, None] * nell_cv, axis=1))

    km_c = km_c_ref[0]
    pe = pe_ref[...]
    know = know + jnp.where(km_c == 0, 0.0, pe)

    km_r = km_r_ref[0]
    s2 = lax.dot_general(q2, know, (((1,), (1,)), ((), ())),
                         preferred_element_type=jnp.float32)
    s2 = jnp.where(km_r == 0, NEG, s2)
    mx2 = jnp.max(s2, axis=1, keepdims=True)
    e2 = jnp.exp(s2 - mx2)
    a2 = e2 / jnp.sum(e2, axis=1, keepdims=True)
    ctx = jnp.dot(a2, know, preferred_element_type=jnp.float32)

    mu = jnp.mean(ctx, axis=1, keepdims=True)
    var = jnp.mean((ctx - mu) ** 2, axis=1, keepdims=True)
    ctx = gamma_ref[...] * (ctx - mu) * lax.rsqrt(var + 1e-5) + beta_ref[...]

    know_ref[0] = know
    ctx_ref[0] = ctx


def _dense(slot, intent, wn_cv, nell_cv, wn_idx, nell_idx, km, wk, wc,
           gamma, beta, pe):
    B, S, D = slot.shape
    km_r = km.reshape(B, 1, S)
    km_c = km.reshape(B, S, 1)
    bspec = lambda shp: pl.BlockSpec((1,) + shp, lambda b: (b,) + (0,) * len(shp))
    full = lambda shp: pl.BlockSpec(shp, lambda b: (0,) * len(shp))
    return pl.pallas_call(
        _dense_body,
        grid=(B,),
        in_specs=[
            bspec((S, D)), bspec((S, D)),
            bspec((S, CC, KB)), bspec((S, CC, KB)),
            bspec((S, CC)), bspec((S, CC)),
            bspec((1, S)), bspec((S, 1)),
            full((2 * D, KB)), full((2 * D, KB)),
            full((1, KB)), full((1, KB)), full((S, KB)),
        ],
        out_specs=[bspec((S, KB)), bspec((S, KB))],
        out_shape=[
            jax.ShapeDtypeStruct((B, S, KB), jnp.float32),
            jax.ShapeDtypeStruct((B, S, KB), jnp.float32),
        ],
        interpret=_INTERPRET,
    )(slot, intent, wn_cv, nell_cv, wn_idx, nell_idx, km_r, km_c, wk, wc,
      gamma.reshape(1, KB), beta.reshape(1, KB), pe)


def kernel(intent_features, slot_features, attention_mask, wn_synset_indexes,
           wn_synset_lengths, nell_entity_indexes, nell_entity_lengths,
           wn_table, nell_table, W_k, W_c, gamma, beta, pos_embed):
    B, S, D = slot_features.shape
    wn_cv = jnp.take(wn_table, wn_synset_indexes, axis=0)
    nell_cv = jnp.take(nell_table, nell_entity_indexes, axis=0)
    km = (wn_synset_lengths + nell_entity_lengths).astype(jnp.int32)
    know, ctx = _dense(slot_features, intent_features, wn_cv, nell_cv,
                       wn_synset_indexes.astype(jnp.int32),
                       nell_entity_indexes.astype(jnp.int32),
                       km, W_k, W_c, gamma, beta, pos_embed)
    return (know, ctx)
